# baseline (speedup 1.0000x reference)
import functools

import jax
import jax.numpy as jnp
from jax import lax
from jax.experimental import pallas as pl
from jax.experimental.pallas import tpu as pltpu
from jax.experimental.pallas import tpu_sc as plsc

_VOCAB = 50257
_HIDDEN = 1024
_B = 4
_S = 2048
_N = _B * _S
_NC = 2
_NS = 16
_NW = _NC * _NS
_P = _S // _NW
_PJ = 8
_CR = _B * _PJ
_NCH = _P // _PJ
_LANES = 16
_NBUF = 2


def _emb_body(ids_hbm, word_hbm, pos_hbm, out_hbm, idsb_v, idx_v, rows_v,
              pos_v, shared_v, gsem0, gsem1, osem0, osem1, psem0, psem1,
              xsem):
  gsems = (gsem0, gsem1)
  osems = (osem0, osem1)
  psems = (psem0, psem1)
  sid = lax.axis_index("s")
  wid = sid * _NC + lax.axis_index("c")
  pos_base = wid * _P

  id_copies = [
      pltpu.async_copy(ids_hbm.at[b, pl.ds(pos_base, _P)],
                       idsb_v.at[pl.ds(b * _P, _P)], psem1)
      for b in range(_B)
  ]
  for cp in id_copies:
    cp.wait()
  lo_half = lax.iota(jnp.int32, _LANES) < _PJ
  for ch in range(_NCH):
    for h in range(_CR // _LANES):
      b0 = 2 * h
      b1 = 2 * h + 1
      va = idsb_v[pl.ds(b0 * _P + ch * _PJ, _LANES)]
      vb = idsb_v[pl.ds(b1 * _P + ch * _PJ - _PJ, _LANES)]
      idx_v[ch, pl.ds(h * _LANES, _LANES)] = jnp.where(lo_half, va, vb)

  def gather(ch):
    return pltpu.async_copy(word_hbm.at[idx_v.at[ch]],
                            rows_v.at[ch % _NBUF], gsems[ch % _NBUF])

  def pload(ch):
    return pltpu.async_copy(
        pos_hbm.at[pl.ds(pos_base + ch * _PJ, _PJ)],
        pos_v.at[ch % 2], psems[ch % 2])

  gathers = [None] * _NBUF
  ploads = [None, None]
  last_stores = []

  gathers[0] = gather(0)
  ploads[0] = pload(0)

  for ch in range(_NCH):
    cur = ch % _NBUF
    nxt = (ch + 1) % _NBUF
    if ch + 1 < _NCH:
      gathers[nxt] = gather(ch + 1)
      ploads[(ch + 1) % 2] = pload(ch + 1)
    gathers[cur].wait()
    ploads[ch % 2].wait()

    def row_body(r, c2):
      for grp in range(_HIDDEN // _LANES):
        sl = pl.ds(grp * _LANES, _LANES)
        p = pos_v[ch % 2, r, sl]
        for b in range(_B):
          rows_v[cur, b * _PJ + r, sl] += p
      return c2

    lax.fori_loop(0, _PJ, row_body, 0)

    for st in last_stores:
      st.wait()
    pltpu.async_copy(rows_v.at[cur], shared_v.at[sid], xsem).wait()
    last_stores = [
        pltpu.async_copy(
            shared_v.at[sid, pl.ds(b * _PJ, _PJ)],
            out_hbm.at[pl.ds(b * _S + pos_base + ch * _PJ, _PJ)],
            osems[ch % 2])
        for b in range(_B)
    ]

  for st in last_stores:
    st.wait()


_mesh = plsc.VectorSubcoreMesh(
    core_axis_name="c", subcore_axis_name="s", num_cores=_NC,
    num_subcores=_NS)

_emb_kernel = functools.partial(
    pl.kernel,
    out_type=jax.ShapeDtypeStruct((_N, _HIDDEN), jnp.float32),
    mesh=_mesh,
    scratch_types=[
        pltpu.VMEM((_B * _P,), jnp.int32),
        pltpu.VMEM((_NCH, _CR), jnp.int32),
        pltpu.VMEM((_NBUF, _CR, _HIDDEN), jnp.float32),
        pltpu.VMEM((2, _PJ, _HIDDEN), jnp.float32),
        pltpu.VMEM_SHARED((_NS, _CR, _HIDDEN), jnp.float32),
        pltpu.SemaphoreType.DMA,
        pltpu.SemaphoreType.DMA,
        pltpu.SemaphoreType.DMA,
        pltpu.SemaphoreType.DMA,
        pltpu.SemaphoreType.DMA,
        pltpu.SemaphoreType.DMA,
        pltpu.SemaphoreType.DMA,
    ],
)(_emb_body)


@jax.jit
def kernel(input_ids, word_embeddings, position_embeddings):
  out = _emb_kernel(input_ids, word_embeddings, position_embeddings)
  return out.reshape(_B, _S, _HIDDEN)


# speedup vs baseline: 1.0275x; 1.0275x over previous
import functools

import jax
import jax.numpy as jnp
from jax import lax
from jax.experimental import pallas as pl
from jax.experimental.pallas import tpu as pltpu
from jax.experimental.pallas import tpu_sc as plsc

_VOCAB = 50257
_HIDDEN = 1024
_B = 4
_S = 2048
_N = _B * _S
_NC = 2
_NS = 16
_NW = _NC * _NS
_P = _S // _NW
_PJ = 8
_CR = _B * _PJ
_NCH = _P // _PJ
_LANES = 16
_NBUF = 2


def _emb_body(ids_hbm, word_hbm, pos_hbm, out_hbm, idsb_v, idx_v, rows_v,
              pos_v, gsem0, gsem1, osem0, osem1, psem0, psem1):
  gsems = (gsem0, gsem1)
  osems = (osem0, osem1)
  psems = (psem0, psem1)
  wid = lax.axis_index("s") * _NC + lax.axis_index("c")
  pos_base = wid * _P

  id_copies = [
      pltpu.async_copy(ids_hbm.at[b, pl.ds(pos_base, _P)],
                       idsb_v.at[pl.ds(b * _P, _P)], psem1)
      for b in range(_B)
  ]
  for cp in id_copies:
    cp.wait()
  lo_half = lax.iota(jnp.int32, _LANES) < _PJ
  for ch in range(_NCH):
    for h in range(_CR // _LANES):
      b0 = 2 * h
      b1 = 2 * h + 1
      va = idsb_v[pl.ds(b0 * _P + ch * _PJ, _LANES)]
      vb = idsb_v[pl.ds(b1 * _P + ch * _PJ - _PJ, _LANES)]
      idx_v[ch, pl.ds(h * _LANES, _LANES)] = jnp.where(lo_half, va, vb)

  def gather(ch):
    return pltpu.async_copy(word_hbm.at[idx_v.at[ch]],
                            rows_v.at[ch % _NBUF], gsems[ch % _NBUF])

  def pload(ch):
    return pltpu.async_copy(
        pos_hbm.at[pl.ds(pos_base + ch * _PJ, _PJ)],
        pos_v.at[ch % 2], psems[ch % 2])

  gathers = [None] * _NBUF
  ploads = [None, None]
  stores = [[None] * _B for _ in range(_NBUF)]

  gathers[0] = gather(0)
  ploads[0] = pload(0)

  for ch in range(_NCH):
    cur = ch % _NBUF
    nxt = (ch + 1) % _NBUF
    if ch + 1 < _NCH:
      for st in stores[nxt]:
        if st is not None:
          st.wait()
      gathers[nxt] = gather(ch + 1)
      ploads[(ch + 1) % 2] = pload(ch + 1)
    gathers[cur].wait()
    ploads[ch % 2].wait()

    def row_body(r, c2):
      for grp in range(_HIDDEN // _LANES):
        sl = pl.ds(grp * _LANES, _LANES)
        p = pos_v[ch % 2, r, sl]
        for b in range(_B):
          rows_v[cur, b * _PJ + r, sl] += p
      return c2

    lax.fori_loop(0, _PJ, row_body, 0)

    for b in range(_B):
      out_off = b * _S + pos_base + ch * _PJ
      stores[cur][b] = pltpu.async_copy(
          rows_v.at[cur, pl.ds(b * _PJ, _PJ)],
          out_hbm.at[pl.ds(out_off, _PJ)], osems[cur])

  for buf in stores:
    for st in buf:
      if st is not None:
        st.wait()


_mesh = plsc.VectorSubcoreMesh(
    core_axis_name="c", subcore_axis_name="s", num_cores=_NC,
    num_subcores=_NS)

_emb_kernel = functools.partial(
    pl.kernel,
    out_type=jax.ShapeDtypeStruct((_N, _HIDDEN), jnp.float32),
    mesh=_mesh,
    scratch_types=[
        pltpu.VMEM((_B * _P,), jnp.int32),
        pltpu.VMEM((_NCH, _CR), jnp.int32),
        pltpu.VMEM((_NBUF, _CR, _HIDDEN), jnp.float32),
        pltpu.VMEM((2, _PJ, _HIDDEN), jnp.float32),
        pltpu.SemaphoreType.DMA,
        pltpu.SemaphoreType.DMA,
        pltpu.SemaphoreType.DMA,
        pltpu.SemaphoreType.DMA,
        pltpu.SemaphoreType.DMA,
        pltpu.SemaphoreType.DMA,
    ],
)(_emb_body)


@jax.jit
def kernel(input_ids, word_embeddings, position_embeddings):
  out = _emb_kernel(input_ids, word_embeddings, position_embeddings)
  return out.reshape(_B, _S, _HIDDEN)


# speedup vs baseline: 1.0358x; 1.0081x over previous
import functools

import jax
import jax.numpy as jnp
from jax import lax
from jax.experimental import pallas as pl
from jax.experimental.pallas import tpu as pltpu
from jax.experimental.pallas import tpu_sc as plsc

_VOCAB = 50257
_HIDDEN = 1024
_B = 4
_S = 2048
_N = _B * _S
_NC = 2
_NS = 16
_NW = _NC * _NS
_P = _S // _NW
_PJ = 8
_CR = _B * _PJ
_NCH = _P // _PJ
_LANES = 16
_NBUF = 3


def _emb_body(ids_hbm, word_hbm, pos_hbm, out_hbm, idsb_v, idx_v, rows_v,
              pos_v, gsem0, gsem1, gsem2, osem0, osem1, osem2, psem0, psem1):
  gsems = (gsem0, gsem1, gsem2)
  osems = (osem0, osem1, osem2)
  psems = (psem0, psem1)
  wid = lax.axis_index("s") * _NC + lax.axis_index("c")
  pos_base = wid * _P

  id_copies = [
      pltpu.async_copy(ids_hbm.at[b, pl.ds(pos_base, _P)],
                       idsb_v.at[pl.ds(b * _P, _P)], psem1)
      for b in range(_B)
  ]
  for cp in id_copies:
    cp.wait()
  lo_half = lax.iota(jnp.int32, _LANES) < _PJ
  for ch in range(_NCH):
    for h in range(_CR // _LANES):
      b0 = 2 * h
      b1 = 2 * h + 1
      va = idsb_v[pl.ds(b0 * _P + ch * _PJ, _LANES)]
      vb = idsb_v[pl.ds(b1 * _P + ch * _PJ - _PJ, _LANES)]
      idx_v[ch, pl.ds(h * _LANES, _LANES)] = jnp.where(lo_half, va, vb)

  def gather(ch):
    return pltpu.async_copy(word_hbm.at[idx_v.at[ch]],
                            rows_v.at[ch % _NBUF], gsems[ch % _NBUF])

  def pload(ch):
    return pltpu.async_copy(
        pos_hbm.at[pl.ds(pos_base + ch * _PJ, _PJ)],
        pos_v.at[ch % 2], psems[ch % 2])

  gathers = [None] * _NBUF
  ploads = [None, None]
  stores = [[None] * _B for _ in range(_NBUF)]

  gathers[0] = gather(0)
  ploads[0] = pload(0)

  for ch in range(_NCH):
    cur = ch % _NBUF
    nxt = (ch + 1) % _NBUF
    if ch + 1 < _NCH:
      for st in stores[nxt]:
        if st is not None:
          st.wait()
      gathers[nxt] = gather(ch + 1)
      ploads[(ch + 1) % 2] = pload(ch + 1)
    gathers[cur].wait()
    ploads[ch % 2].wait()

    def row_body(r, c2):
      for grp in range(_HIDDEN // _LANES):
        sl = pl.ds(grp * _LANES, _LANES)
        p = pos_v[ch % 2, r, sl]
        for b in range(_B):
          rows_v[cur, b * _PJ + r, sl] += p
      return c2

    lax.fori_loop(0, _PJ, row_body, 0)

    for b in range(_B):
      out_off = b * _S + pos_base + ch * _PJ
      stores[cur][b] = pltpu.async_copy(
          rows_v.at[cur, pl.ds(b * _PJ, _PJ)],
          out_hbm.at[pl.ds(out_off, _PJ)], osems[cur])

  for buf in stores:
    for st in buf:
      if st is not None:
        st.wait()


_mesh = plsc.VectorSubcoreMesh(
    core_axis_name="c", subcore_axis_name="s", num_cores=_NC,
    num_subcores=_NS)

_emb_kernel = functools.partial(
    pl.kernel,
    out_type=jax.ShapeDtypeStruct((_N, _HIDDEN), jnp.float32),
    mesh=_mesh,
    scratch_types=[
        pltpu.VMEM((_B * _P,), jnp.int32),
        pltpu.VMEM((_NCH, _CR), jnp.int32),
        pltpu.VMEM((_NBUF, _CR, _HIDDEN), jnp.float32),
        pltpu.VMEM((2, _PJ, _HIDDEN), jnp.float32),
        pltpu.SemaphoreType.DMA,
        pltpu.SemaphoreType.DMA,
        pltpu.SemaphoreType.DMA,
        pltpu.SemaphoreType.DMA,
        pltpu.SemaphoreType.DMA,
        pltpu.SemaphoreType.DMA,
        pltpu.SemaphoreType.DMA,
        pltpu.SemaphoreType.DMA,
    ],
)(_emb_body)


@jax.jit
def kernel(input_ids, word_embeddings, position_embeddings):
  out = _emb_kernel(input_ids, word_embeddings, position_embeddings)
  return out.reshape(_B, _S, _HIDDEN)
